# baseline (speedup 1.0000x reference)
import functools

import jax
import jax.numpy as jnp
from jax import lax
from jax.experimental import pallas as pl
from jax.experimental.pallas import tpu as pltpu
from jax.experimental.pallas import tpu_sc as plsc

T = 2048
D = 768
F = 2048
E = 8
TOP_K = 2

RT = 512
BT = 256
EPAD = 128
P = T * TOP_K + E * BT
NT = P // BT

NW = 32
DISP_CH = 48
COMB_CH = 32


def _router_body(x_ref, wgt_ref, comb_ref, sel_ref, topw_ref, aux_ref, acc_ref):
    i = pl.program_id(0)
    nsteps = pl.num_programs(0)
    logits = jnp.dot(x_ref[...], wgt_ref[...], preferred_element_type=jnp.float32)
    lane = jax.lax.broadcasted_iota(jnp.int32, logits.shape, 1)
    lm = jnp.where(lane < E, logits, -jnp.inf)
    m = jnp.max(lm, axis=1, keepdims=True)
    p = jnp.exp(lm - m)
    s = jnp.sum(p, axis=1, keepdims=True)
    z = jnp.log(s) + m

    @pl.when(i == 0)
    def _():
        acc_ref[0, 0] = 0.0

    acc_ref[0, 0] += jnp.sum(z * z)

    probs = p / s
    big = jnp.int32(999)
    p1 = jnp.max(probs, axis=1, keepdims=True)
    a1 = jnp.min(jnp.where(probs == p1, lane, big), axis=1, keepdims=True)
    probs2 = jnp.where(lane == a1, -1.0, probs)
    p2 = jnp.max(probs2, axis=1, keepdims=True)
    a2 = jnp.min(jnp.where(probs2 == p2, lane, big), axis=1, keepdims=True)
    wsum = p1 + p2
    w1 = p1 / wsum
    w2 = p2 / wsum
    comb = jnp.where(lane == a1, w1, 0.0) + jnp.where(lane == a2, w2, 0.0)
    comb_ref[...] = comb[:, :E]
    sel = jnp.where(lane == 0, a1, jnp.where(lane == 1, a2, 0))
    sel_ref[...] = sel[:, :TOP_K]
    topw = jnp.where(lane == 0, w1, jnp.where(lane == 1, w2, 0.0))
    topw_ref[...] = topw[:, :TOP_K]

    @pl.when(i == nsteps - 1)
    def _():
        aux_ref[0, 0] = acc_ref[0, 0] * (0.001 / T)


def _router(x, w_gate):
    wgt = jnp.zeros((D, EPAD), jnp.float32).at[:, :E].set(w_gate.T)
    return pl.pallas_call(
        _router_body,
        grid=(T // RT,),
        in_specs=[
            pl.BlockSpec((RT, D), lambda i: (i, 0)),
            pl.BlockSpec((D, EPAD), lambda i: (0, 0)),
        ],
        out_specs=[
            pl.BlockSpec((RT, E), lambda i: (i, 0)),
            pl.BlockSpec((RT, TOP_K), lambda i: (i, 0)),
            pl.BlockSpec((RT, TOP_K), lambda i: (i, 0)),
            pl.BlockSpec((1, 1), lambda i: (0, 0), memory_space=pltpu.SMEM),
        ],
        out_shape=[
            jax.ShapeDtypeStruct((T, E), jnp.float32),
            jax.ShapeDtypeStruct((T, TOP_K), jnp.int32),
            jax.ShapeDtypeStruct((T, TOP_K), jnp.float32),
            jax.ShapeDtypeStruct((1, 1), jnp.float32),
        ],
        scratch_shapes=[pltpu.SMEM((1, 1), jnp.float32)],
    )(x, wgt)


def _routing_plan(sel, topw):
    ep = sel.reshape(-1)
    oh = (ep[:, None] == jnp.arange(E)[None, :]).astype(jnp.int32)
    cum = jnp.cumsum(oh, axis=0)
    pos_within = jnp.sum((cum - oh) * oh, axis=1)
    counts = cum[-1]
    padded = ((counts + BT - 1) // BT) * BT
    offs = jnp.concatenate([jnp.zeros((1,), jnp.int32),
                            jnp.cumsum(padded).astype(jnp.int32)])
    pos = offs[ep] + pos_within
    tok = jnp.zeros((P,), jnp.int32).at[pos].set(jnp.arange(T * TOP_K) // TOP_K)
    wslot = jnp.zeros((P, 1), jnp.float32).at[pos, 0].set(topw.reshape(-1))
    tile_start = jnp.arange(NT, dtype=jnp.int32) * BT
    tile_expert = jnp.clip(
        jnp.sum((tile_start[:, None] >= offs[1:][None, :]).astype(jnp.int32), axis=1),
        0, E - 1)
    n_active = offs[E] // BT
    meta = jnp.concatenate([tile_expert, n_active[None]]).astype(jnp.int32)
    return pos.astype(jnp.int32), tok, wslot, meta


def _dispatch(x, tok):
    per_w = P // NW
    mesh = plsc.VectorSubcoreMesh(core_axis_name="c", subcore_axis_name="s")

    @functools.partial(
        pl.kernel,
        mesh=mesh,
        out_type=jax.ShapeDtypeStruct((P, D), jnp.float32),
        scratch_types=[
            pltpu.VMEM((per_w,), jnp.int32),
            pltpu.VMEM((DISP_CH, D), jnp.float32),
            pltpu.SemaphoreType.DMA,
        ],
    )
    def disp(x_hbm, tok_hbm, xs_hbm, idx_v, rows_v, sem):
        wid = lax.axis_index("s") * 2 + lax.axis_index("c")
        base = wid * per_w
        pltpu.sync_copy(tok_hbm.at[pl.ds(base, per_w)], idx_v)
        for c in range(per_w // DISP_CH):
            pltpu.async_copy(
                x_hbm.at[idx_v.at[pl.ds(c * DISP_CH, DISP_CH)]], rows_v, sem
            ).wait()
            pltpu.sync_copy(rows_v, xs_hbm.at[pl.ds(base + c * DISP_CH, DISP_CH)])

    return disp(x, tok)


def _gmm_body(meta_ref, xs_ref, wg_ref, wu_ref, wd_ref, ws_ref, ys_ref):
    i = pl.program_id(0)

    @pl.when(i < meta_ref[NT])
    def _():
        x = xs_ref[...]
        g = jnp.dot(x, wg_ref[0], preferred_element_type=jnp.float32)
        u = jnp.dot(x, wu_ref[0], preferred_element_type=jnp.float32)
        h = (g * jax.nn.sigmoid(g)) * u
        y = jnp.dot(h, wd_ref[0], preferred_element_type=jnp.float32)
        ys_ref[...] = y * ws_ref[...]


def _gmm(meta, xs, wslot, gate_w, up_w, down_w):
    grid_spec = pltpu.PrefetchScalarGridSpec(
        num_scalar_prefetch=1,
        grid=(NT,),
        in_specs=[
            pl.BlockSpec((BT, D), lambda i, m: (i, 0)),
            pl.BlockSpec((1, D, F), lambda i, m: (m[i], 0, 0)),
            pl.BlockSpec((1, D, F), lambda i, m: (m[i], 0, 0)),
            pl.BlockSpec((1, F, D), lambda i, m: (m[i], 0, 0)),
            pl.BlockSpec((BT, 1), lambda i, m: (i, 0)),
        ],
        out_specs=pl.BlockSpec((BT, D), lambda i, m: (i, 0)),
    )
    return pl.pallas_call(
        _gmm_body,
        grid_spec=grid_spec,
        out_shape=jax.ShapeDtypeStruct((P, D), jnp.float32),
        compiler_params=pltpu.CompilerParams(
            dimension_semantics=("arbitrary",),
            vmem_limit_bytes=110 * 1024 * 1024,
        ),
    )(meta, xs, gate_w, up_w, down_w, wslot)


def _combine(ys, pos_flat):
    per_t = T // NW
    n_ch = per_t // COMB_CH
    mesh = plsc.VectorSubcoreMesh(core_axis_name="c", subcore_axis_name="s")

    @functools.partial(
        pl.kernel,
        mesh=mesh,
        out_type=jax.ShapeDtypeStruct((T, D), jnp.float32),
        scratch_types=[
            pltpu.VMEM((TOP_K * per_t,), jnp.int32),
            pltpu.VMEM((TOP_K * COMB_CH, D), jnp.float32),
            pltpu.VMEM((COMB_CH, D), jnp.float32),
            pltpu.SemaphoreType.DMA,
        ],
    )
    def comb(ys_hbm, pos_hbm, out_hbm, idx_v, rows_v, acc_v, sem):
        wid = lax.axis_index("s") * 2 + lax.axis_index("c")
        base = wid * per_t
        pltpu.sync_copy(pos_hbm.at[pl.ds(base * TOP_K, per_t * TOP_K)], idx_v)
        for c in range(n_ch):
            pltpu.async_copy(
                ys_hbm.at[idx_v.at[pl.ds(c * COMB_CH * TOP_K, COMB_CH * TOP_K)]],
                rows_v, sem,
            ).wait()

            def body(j, _):
                for v in range(D // 16):
                    sl = pl.ds(16 * v, 16)
                    acc_v[j, sl] = rows_v[2 * j, sl] + rows_v[2 * j + 1, sl]
                return 0

            lax.fori_loop(0, COMB_CH, body, 0)
            pltpu.sync_copy(acc_v, out_hbm.at[pl.ds(base + c * COMB_CH, COMB_CH)])

    return comb(ys, pos_flat)


def kernel(hidden_states, w_gate, w_u, w_g, w_d):
    gate_w, up_w, down_w = w_u, w_g, w_d
    b, s, d = hidden_states.shape
    x = hidden_states.reshape(-1, d)
    comb, sel, topw, aux = _router(x, w_gate)
    pos, tok, wslot, meta = _routing_plan(sel, topw)
    xs = _dispatch(x, tok)
    ys = _gmm(meta, xs, wslot, gate_w, up_w, down_w)
    final = _combine(ys, pos)
    return final.reshape(b, s, d), aux.reshape(())


# speedup vs baseline: 1.0462x; 1.0462x over previous
import functools

import jax
import jax.numpy as jnp
from jax import lax
from jax.experimental import pallas as pl
from jax.experimental.pallas import tpu as pltpu
from jax.experimental.pallas import tpu_sc as plsc

T = 2048
D = 768
F = 2048
E = 8
TOP_K = 2

RT = 512
BT = 256
EPAD = 128
P = T * TOP_K + E * BT
NT = P // BT

NW = 32
DISP_CH = 48
COMB_CH = 32


def _router_body(x_ref, wgt_ref, comb_ref, sel_ref, topw_ref, aux_ref, acc_ref):
    i = pl.program_id(0)
    nsteps = pl.num_programs(0)
    logits = jnp.dot(x_ref[...], wgt_ref[...], preferred_element_type=jnp.float32)
    lane = jax.lax.broadcasted_iota(jnp.int32, logits.shape, 1)
    lm = jnp.where(lane < E, logits, -jnp.inf)
    m = jnp.max(lm, axis=1, keepdims=True)
    p = jnp.exp(lm - m)
    s = jnp.sum(p, axis=1, keepdims=True)
    z = jnp.log(s) + m

    @pl.when(i == 0)
    def _():
        acc_ref[0, 0] = 0.0

    acc_ref[0, 0] += jnp.sum(z * z)

    probs = p / s
    big = jnp.int32(999)
    p1 = jnp.max(probs, axis=1, keepdims=True)
    a1 = jnp.min(jnp.where(probs == p1, lane, big), axis=1, keepdims=True)
    probs2 = jnp.where(lane == a1, -1.0, probs)
    p2 = jnp.max(probs2, axis=1, keepdims=True)
    a2 = jnp.min(jnp.where(probs2 == p2, lane, big), axis=1, keepdims=True)
    wsum = p1 + p2
    w1 = p1 / wsum
    w2 = p2 / wsum
    comb = jnp.where(lane == a1, w1, 0.0) + jnp.where(lane == a2, w2, 0.0)
    comb_ref[...] = comb[:, :E]
    sel = jnp.where(lane == 0, a1, jnp.where(lane == 1, a2, 0))
    sel_ref[...] = sel[:, :TOP_K]
    topw = jnp.where(lane == 0, w1, jnp.where(lane == 1, w2, 0.0))
    topw_ref[...] = topw[:, :TOP_K]

    @pl.when(i == nsteps - 1)
    def _():
        aux_ref[0, 0] = acc_ref[0, 0] * (0.001 / T)


def _router(x, w_gate):
    wgt = jnp.zeros((D, EPAD), jnp.float32).at[:, :E].set(w_gate.T)
    return pl.pallas_call(
        _router_body,
        grid=(T // RT,),
        in_specs=[
            pl.BlockSpec((RT, D), lambda i: (i, 0)),
            pl.BlockSpec((D, EPAD), lambda i: (0, 0)),
        ],
        out_specs=[
            pl.BlockSpec((RT, E), lambda i: (i, 0)),
            pl.BlockSpec((RT, TOP_K), lambda i: (i, 0)),
            pl.BlockSpec((RT, TOP_K), lambda i: (i, 0)),
            pl.BlockSpec((1, 1), lambda i: (0, 0), memory_space=pltpu.SMEM),
        ],
        out_shape=[
            jax.ShapeDtypeStruct((T, E), jnp.float32),
            jax.ShapeDtypeStruct((T, TOP_K), jnp.int32),
            jax.ShapeDtypeStruct((T, TOP_K), jnp.float32),
            jax.ShapeDtypeStruct((1, 1), jnp.float32),
        ],
        scratch_shapes=[pltpu.SMEM((1, 1), jnp.float32)],
    )(x, wgt)


def _routing_plan(sel, topw):
    ep = sel.reshape(-1)
    oh = (ep[:, None] == jnp.arange(E)[None, :]).astype(jnp.int32)
    cum = jnp.cumsum(oh, axis=0)
    pos_within = jnp.sum((cum - oh) * oh, axis=1)
    counts = cum[-1]
    padded = ((counts + BT - 1) // BT) * BT
    offs = jnp.concatenate([jnp.zeros((1,), jnp.int32),
                            jnp.cumsum(padded).astype(jnp.int32)])
    pos = offs[ep] + pos_within
    pos_t = pos.reshape(T, TOP_K).T.reshape(-1)
    tok = jnp.zeros((P,), jnp.int32).at[pos].set(jnp.arange(T * TOP_K) // TOP_K)
    wslot = jnp.zeros((P, 1), jnp.float32).at[pos, 0].set(topw.reshape(-1))
    tile_start = jnp.arange(NT, dtype=jnp.int32) * BT
    tile_expert = jnp.clip(
        jnp.sum((tile_start[:, None] >= offs[1:][None, :]).astype(jnp.int32), axis=1),
        0, E - 1)
    n_active = offs[E] // BT
    meta = jnp.concatenate([tile_expert, n_active[None]]).astype(jnp.int32)
    return pos_t.astype(jnp.int32), tok, wslot, meta


def _dispatch(x, tok):
    per_w = P // NW
    ch = DISP_CH
    n_ch = per_w // ch
    mesh = plsc.VectorSubcoreMesh(core_axis_name="c", subcore_axis_name="s")

    @functools.partial(
        pl.kernel,
        mesh=mesh,
        out_type=jax.ShapeDtypeStruct((P, D), jnp.float32),
        scratch_types=[
            pltpu.VMEM((per_w,), jnp.int32),
            pltpu.VMEM((ch, D), jnp.float32),
            pltpu.VMEM((ch, D), jnp.float32),
            pltpu.VMEM((ch, D), jnp.float32),
            pltpu.SemaphoreType.DMA,
            pltpu.SemaphoreType.DMA,
            pltpu.SemaphoreType.DMA,
            pltpu.SemaphoreType.DMA,
            pltpu.SemaphoreType.DMA,
            pltpu.SemaphoreType.DMA,
        ],
    )
    def disp(x_hbm, tok_hbm, xs_hbm, idx_v, b0, b1, b2, g0, g1, g2, w0, w1, w2):
        wid = lax.axis_index("s") * 2 + lax.axis_index("c")
        base = wid * per_w
        pltpu.sync_copy(tok_hbm.at[pl.ds(base, per_w)], idx_v)
        bufs = (b0, b1, b2)
        gsems = (g0, g1, g2)
        wsems = (w0, w1, w2)
        gathers = {}
        writes = {}
        for c in range(min(3, n_ch)):
            gathers[c] = pltpu.async_copy(
                x_hbm.at[idx_v.at[pl.ds(c * ch, ch)]], bufs[c], gsems[c])
        for c in range(n_ch):
            k = c % 3
            gathers[c].wait()
            writes[c] = pltpu.async_copy(
                bufs[k], xs_hbm.at[pl.ds(base + c * ch, ch)], wsems[k])
            if c + 3 < n_ch:
                writes[c].wait()
                gathers[c + 3] = pltpu.async_copy(
                    x_hbm.at[idx_v.at[pl.ds((c + 3) * ch, ch)]], bufs[k], gsems[k])
        for c in sorted(writes)[-3:]:
            if c + 3 >= n_ch:
                writes[c].wait()

    return disp(x, tok)


def _gmm_body(meta_ref, xs_ref, wg_ref, wu_ref, wd_ref, ws_ref, ys_ref):
    i = pl.program_id(0)

    @pl.when(i < meta_ref[NT])
    def _():
        x = xs_ref[...]
        g = jnp.dot(x, wg_ref[0], preferred_element_type=jnp.float32)
        u = jnp.dot(x, wu_ref[0], preferred_element_type=jnp.float32)
        h = (g * jax.nn.sigmoid(g)) * u
        y = jnp.dot(h, wd_ref[0], preferred_element_type=jnp.float32)
        ys_ref[...] = y * ws_ref[...]


def _gmm(meta, xs, wslot, gate_w, up_w, down_w):
    grid_spec = pltpu.PrefetchScalarGridSpec(
        num_scalar_prefetch=1,
        grid=(NT,),
        in_specs=[
            pl.BlockSpec((BT, D), lambda i, m: (i, 0)),
            pl.BlockSpec((1, D, F), lambda i, m: (m[i], 0, 0)),
            pl.BlockSpec((1, D, F), lambda i, m: (m[i], 0, 0)),
            pl.BlockSpec((1, F, D), lambda i, m: (m[i], 0, 0)),
            pl.BlockSpec((BT, 1), lambda i, m: (i, 0)),
        ],
        out_specs=pl.BlockSpec((BT, D), lambda i, m: (i, 0)),
    )
    return pl.pallas_call(
        _gmm_body,
        grid_spec=grid_spec,
        out_shape=jax.ShapeDtypeStruct((P, D), jnp.float32),
        compiler_params=pltpu.CompilerParams(
            dimension_semantics=("arbitrary",),
            vmem_limit_bytes=110 * 1024 * 1024,
        ),
    )(meta, xs, gate_w, up_w, down_w, wslot)


def _combine(ys, pos_t):
    per_t = T // NW
    ch = COMB_CH
    n_ch = per_t // ch
    mesh = plsc.VectorSubcoreMesh(core_axis_name="c", subcore_axis_name="s")

    @functools.partial(
        pl.kernel,
        mesh=mesh,
        out_type=jax.ShapeDtypeStruct((T, D), jnp.float32),
        scratch_types=[
            pltpu.VMEM((per_t,), jnp.int32),
            pltpu.VMEM((per_t,), jnp.int32),
            pltpu.VMEM((ch, D), jnp.float32),
            pltpu.VMEM((ch, D), jnp.float32),
            pltpu.VMEM((ch, D), jnp.float32),
            pltpu.VMEM((ch, D), jnp.float32),
            pltpu.SemaphoreType.DMA,
            pltpu.SemaphoreType.DMA,
            pltpu.SemaphoreType.DMA,
            pltpu.SemaphoreType.DMA,
            pltpu.SemaphoreType.DMA,
            pltpu.SemaphoreType.DMA,
        ],
    )
    def comb(ys_hbm, pos_hbm, out_hbm, idx_a, idx_b,
             a0, bb0, a1, bb1, sa0, sb0, sa1, sb1, w0, w1):
        wid = lax.axis_index("s") * 2 + lax.axis_index("c")
        tbase = wid * per_t
        pltpu.sync_copy(pos_hbm.at[pl.ds(tbase, per_t)], idx_a)
        pltpu.sync_copy(pos_hbm.at[pl.ds(T + tbase, per_t)], idx_b)
        abufs = (a0, a1)
        bbufs = (bb0, bb1)
        gas = {}
        gbs = {}
        for c in range(n_ch):
            gas[c] = pltpu.async_copy(
                ys_hbm.at[idx_a.at[pl.ds(c * ch, ch)]], abufs[c], (sa0, sa1)[c])
            gbs[c] = pltpu.async_copy(
                ys_hbm.at[idx_b.at[pl.ds(c * ch, ch)]], bbufs[c], (sb0, sb1)[c])
        wr = {}
        for c in range(n_ch):
            a, bb = abufs[c], bbufs[c]
            gas[c].wait()
            gbs[c].wait()

            def body(j, _, a=a, bb=bb):
                for v in range(D // 16):
                    sl = pl.ds(16 * v, 16)
                    a[j, sl] += bb[j, sl]
                return 0

            lax.fori_loop(0, ch, body, 0)
            wr[c] = pltpu.async_copy(
                a, out_hbm.at[pl.ds(tbase + c * ch, ch)], (w0, w1)[c])
        for c in range(n_ch):
            wr[c].wait()

    return comb(ys, pos_t)


def kernel(hidden_states, w_gate, w_u, w_g, w_d):
    gate_w, up_w, down_w = w_u, w_g, w_d
    b, s, d = hidden_states.shape
    x = hidden_states.reshape(-1, d)
    comb, sel, topw, aux = _router(x, w_gate)
    pos, tok, wslot, meta = _routing_plan(sel, topw)
    xs = _dispatch(x, tok)
    ys = _gmm(meta, xs, wslot, gate_w, up_w, down_w)
    final = _combine(ys, pos)
    return final.reshape(b, s, d), aux.reshape(())
